# baseline (speedup 1.0000x reference)
import jax
import jax.numpy as jnp
from jax import lax
from jax.experimental import pallas as pl
from jax.experimental.pallas import tpu as pltpu
from jax.experimental.pallas import tpu_sc as plsc

NUM_ACTIONS = 6
ACTION_DIM = 32
BATCH = 16384
HIST = 200

NC = 2
NS = 16
NW = NC * NS
L = 16

TUP = 4
ROW_T = HIST // TUP
G = 8
CHUNK_A = G * HIST
CHUNK_T = G * ROW_T
ROW_W = TUP * ACTION_DIM
OUT_W = HIST * ACTION_DIM
NCHUNKS = BATCH // G
CPW = NCHUNKS // NW
TVECS = CHUNK_T // L


def _sc_body(act_hbm, ptab_hbm, out_hbm, lut_s, a0_v, a1_v, t0_v, t1_v,
             r0_v, r1_v, is0, is1, gs0, gs1, ws0, ws1):
    wid = lax.axis_index("s") * NC + lax.axis_index("c")
    base = wid * CPW
    i16 = lax.iota(jnp.int32, 16)
    rvec = i16 % G
    cvec = (i16 // G) * TUP
    acts, tidx, rows = (a0_v, a1_v), (t0_v, t1_v), (r0_v, r1_v)
    isem, gsem, wsem = (is0, is1), (gs0, gs1), (ws0, ws1)

    @pl.when(lax.axis_index("s") == 0)
    def _():
        pltpu.sync_copy(ptab_hbm, lut_s)

    plsc.subcore_barrier()

    def fire_idx(i, b):
        pltpu.async_copy(act_hbm.at[pl.ds((base + i) * G, G)], acts[b], isem[b])

    def drain_idx(b):
        pltpu.make_async_copy(act_hbm.at[pl.ds(0, G)], acts[b], isem[b]).wait()

    def fire_gathers(b):
        for j in range(TVECS):
            col = cvec + j * (2 * TUP)
            a0 = plsc.load_gather(acts[b], [rvec, col])
            a1 = plsc.load_gather(acts[b], [rvec, col + 1])
            a2 = plsc.load_gather(acts[b], [rvec, col + 2])
            a3 = plsc.load_gather(acts[b], [rvec, col + 3])
            idx = ((a0 * NUM_ACTIONS + a1) * NUM_ACTIONS + a2) * NUM_ACTIONS + a3
            tidx[b][pl.ds(j * L, L)] = idx
        for t in range(ROW_T):
            pltpu.async_copy(
                lut_s.at[tidx[b].at[pl.ds(t * G, G)]],
                rows[b].at[pl.ds(t * G, G)],
                gsem[b],
            )

    def drain_gathers(b):
        pltpu.make_async_copy(
            ptab_hbm.at[pl.ds(0, CHUNK_T)], rows[b], gsem[b]
        ).wait()

    def fire_write(i, b):
        row0 = (base + i) * G
        for t in range(ROW_T):
            pltpu.async_copy(
                rows[b].at[pl.ds(t * G, G)],
                out_hbm.at[pl.ds(row0, G), pl.ds(t * ROW_W, ROW_W)],
                wsem[b],
            )

    def drain_write(b):
        pltpu.make_async_copy(
            ptab_hbm.at[pl.ds(0, CHUNK_T)], rows[b], wsem[b]
        ).wait()

    def slot(i, b, first, last):
        @pl.when(jnp.logical_not(first))
        def _():
            drain_write(b)
        drain_idx(b)
        fire_gathers(b)
        @pl.when(jnp.logical_not(last))
        def _():
            fire_idx(i + 2, b)
        @pl.when(i > 0)
        def _():
            drain_gathers(1 - b)
            fire_write(i - 1, 1 - b)

    fire_idx(0, 0)
    fire_idx(1, 1)

    @pl.loop(0, CPW, step=2)
    def _pair(c0):
        slot(c0, 0, c0 == 0, c0 + 2 >= CPW)
        slot(c0 + 1, 1, c0 == 0, c0 + 3 >= CPW)

    drain_gathers((CPW - 1) % 2)
    fire_write(CPW - 1, (CPW - 1) % 2)
    drain_write(0)
    drain_write(1)


def kernel(action, table):
    aidx = jnp.arange(NUM_ACTIONS**TUP, dtype=jnp.int32)
    parts = []
    for k in range(TUP):
        ak = (aidx // (NUM_ACTIONS ** (TUP - 1 - k))) % NUM_ACTIONS
        parts.append(jnp.take(table, ak, axis=0))
    ptab = jnp.concatenate(parts, axis=1)

    kfn = pl.kernel(
        _sc_body,
        out_type=jax.ShapeDtypeStruct((BATCH, OUT_W), jnp.float32),
        mesh=plsc.VectorSubcoreMesh(core_axis_name="c", subcore_axis_name="s"),
        compiler_params=pltpu.CompilerParams(
            needs_layout_passes=False, use_tc_tiling_on_sc=True
        ),
        scratch_types=[
            pltpu.VMEM_SHARED((NUM_ACTIONS**TUP, ROW_W), jnp.float32),
            pltpu.VMEM((G, HIST), jnp.int32),
            pltpu.VMEM((G, HIST), jnp.int32),
            pltpu.VMEM((CHUNK_T,), jnp.int32),
            pltpu.VMEM((CHUNK_T,), jnp.int32),
            pltpu.VMEM((CHUNK_T, ROW_W), jnp.float32),
            pltpu.VMEM((CHUNK_T, ROW_W), jnp.float32),
            pltpu.SemaphoreType.DMA,
            pltpu.SemaphoreType.DMA,
            pltpu.SemaphoreType.DMA,
            pltpu.SemaphoreType.DMA,
            pltpu.SemaphoreType.DMA,
            pltpu.SemaphoreType.DMA,
        ],
    )
    return kfn(action, ptab)


# speedup vs baseline: 1.0025x; 1.0025x over previous
import jax
import jax.numpy as jnp
from jax import lax
from jax.experimental import pallas as pl
from jax.experimental.pallas import tpu as pltpu
from jax.experimental.pallas import tpu_sc as plsc

NUM_ACTIONS = 6
ACTION_DIM = 32
BATCH = 16384
HIST = 200

NC = 2
NS = 16
NW = NC * NS
L = 16

TUP = 4
ROW_T = HIST // TUP
G = 8
CHUNK_A = G * HIST
CHUNK_T = G * ROW_T
ROW_W = TUP * ACTION_DIM
OUT_W = HIST * ACTION_DIM
NCHUNKS = BATCH // G
CPW = NCHUNKS // NW
TVECS = CHUNK_T // L


LPW = NUM_ACTIONS**TUP // NS


def _sc_body(act_hbm, table_hbm, out_hbm, lut_s, tab_v, a0_v, a1_v,
             t0_v, t1_v, r0_v, r1_v, is0, is1, gs0, gs1, ws0, ws1):
    sid = lax.axis_index("s")
    wid = sid * NC + lax.axis_index("c")
    base = wid * CPW
    i16 = lax.iota(jnp.int32, 16)
    perm16 = (i16 % G) * HIST + (i16 // G) * TUP
    acts, tidx, rows = (a0_v, a1_v), (t0_v, t1_v), (r0_v, r1_v)
    isem, gsem, wsem = (is0, is1), (gs0, gs1), (ws0, ws1)

    pltpu.sync_copy(table_hbm, tab_v)

    @pl.loop(0, LPW)
    def _lut(m):
        n = sid * LPW + m
        a = (n // (NUM_ACTIONS**3), (n // (NUM_ACTIONS**2)) % NUM_ACTIONS,
             (n // NUM_ACTIONS) % NUM_ACTIONS, n % NUM_ACTIONS)
        for h in range(2 * TUP):
            av = lax.broadcast_in_dim(a[h // 2], (L,), ())
            col = i16 + (h % 2) * L
            r0_v[m, pl.ds(h * L, L)] = plsc.load_gather(tab_v, [av, col])

    pltpu.sync_copy(r0_v.at[pl.ds(0, LPW)], lut_s.at[pl.ds(sid * LPW, LPW)])
    plsc.subcore_barrier()

    def fire_idx(i, b):
        pltpu.async_copy(act_hbm.at[base + i], acts[b], isem[b])

    def drain_idx(b):
        pltpu.make_async_copy(act_hbm.at[0], acts[b], isem[b]).wait()

    def fire_gathers(b):
        for j in range(TVECS):
            pos = perm16 + j * (2 * TUP)
            a0 = plsc.load_gather(acts[b], [pos])
            a1 = plsc.load_gather(acts[b], [pos + 1])
            a2 = plsc.load_gather(acts[b], [pos + 2])
            a3 = plsc.load_gather(acts[b], [pos + 3])
            idx = ((a0 * NUM_ACTIONS + a1) * NUM_ACTIONS + a2) * NUM_ACTIONS + a3
            tidx[b][pl.ds(j * L, L)] = idx
        for t in range(ROW_T):
            pltpu.async_copy(
                lut_s.at[tidx[b].at[pl.ds(t * G, G)]],
                rows[b].at[pl.ds(t * G, G)],
                gsem[b],
            )

    def drain_gathers(b):
        pltpu.make_async_copy(
            lut_s.at[pl.ds(0, CHUNK_T)], rows[b], gsem[b]
        ).wait()

    def fire_write(i, b):
        row0 = (base + i) * G
        for t in range(ROW_T):
            pltpu.async_copy(
                rows[b].at[pl.ds(t * G, G)],
                out_hbm.at[pl.ds(row0, G), pl.ds(t * ROW_W, ROW_W)],
                wsem[b],
            )

    def drain_write(b):
        pltpu.make_async_copy(
            lut_s.at[pl.ds(0, CHUNK_T)], rows[b], wsem[b]
        ).wait()

    def slot(i, b, first, last):
        @pl.when(jnp.logical_not(first))
        def _():
            drain_write(b)
        drain_idx(b)
        fire_gathers(b)
        @pl.when(jnp.logical_not(last))
        def _():
            fire_idx(i + 2, b)
        @pl.when(i > 0)
        def _():
            drain_gathers(1 - b)
            fire_write(i - 1, 1 - b)

    fire_idx(0, 0)
    fire_idx(1, 1)

    @pl.loop(0, CPW, step=2)
    def _pair(c0):
        slot(c0, 0, c0 == 0, c0 + 2 >= CPW)
        slot(c0 + 1, 1, c0 == 0, c0 + 3 >= CPW)

    drain_gathers((CPW - 1) % 2)
    fire_write(CPW - 1, (CPW - 1) % 2)
    drain_write(0)
    drain_write(1)


def kernel(action, table):
    act2 = action.reshape(NCHUNKS, CHUNK_A)
    kfn = pl.kernel(
        _sc_body,
        out_type=jax.ShapeDtypeStruct((BATCH, OUT_W), jnp.float32),
        mesh=plsc.VectorSubcoreMesh(core_axis_name="c", subcore_axis_name="s"),
        compiler_params=pltpu.CompilerParams(
            needs_layout_passes=False, use_tc_tiling_on_sc=True
        ),
        scratch_types=[
            pltpu.VMEM_SHARED((NUM_ACTIONS**TUP, ROW_W), jnp.float32),
            pltpu.VMEM((NUM_ACTIONS, ACTION_DIM), jnp.float32),
            pltpu.VMEM((CHUNK_A,), jnp.int32),
            pltpu.VMEM((CHUNK_A,), jnp.int32),
            pltpu.VMEM((CHUNK_T,), jnp.int32),
            pltpu.VMEM((CHUNK_T,), jnp.int32),
            pltpu.VMEM((CHUNK_T, ROW_W), jnp.float32),
            pltpu.VMEM((CHUNK_T, ROW_W), jnp.float32),
            pltpu.SemaphoreType.DMA,
            pltpu.SemaphoreType.DMA,
            pltpu.SemaphoreType.DMA,
            pltpu.SemaphoreType.DMA,
            pltpu.SemaphoreType.DMA,
            pltpu.SemaphoreType.DMA,
        ],
    )
    return kfn(act2, table)


# speedup vs baseline: 1.0174x; 1.0149x over previous
import jax
import jax.numpy as jnp
from jax import lax
from jax.experimental import pallas as pl
from jax.experimental.pallas import tpu as pltpu
from jax.experimental.pallas import tpu_sc as plsc

NUM_ACTIONS = 6
ACTION_DIM = 32
BATCH = 16384
HIST = 200

NC = 2
NS = 16
NW = NC * NS
L = 16

TUP = 4
ROW_T = HIST // TUP
G = 8
CHUNK_A = G * HIST
CHUNK_T = G * ROW_T
ROW_W = TUP * ACTION_DIM
OUT_W = HIST * ACTION_DIM
NCHUNKS = BATCH // G
CPW = NCHUNKS // NW
TVECS = CHUNK_T // L


def _sc_body(act_hbm, ptab_hbm, out_hbm, lut_s, a0_v, a1_v, t0_v, t1_v,
             r0_v, r1_v, is0, is1, gs0, gs1, ws0, ws1):
    wid = lax.axis_index("s") * NC + lax.axis_index("c")
    base = wid * CPW
    i16 = lax.iota(jnp.int32, 16)
    perm16 = (i16 % G) * HIST + (i16 // G) * TUP
    acts, tidx, rows = (a0_v, a1_v), (t0_v, t1_v), (r0_v, r1_v)
    isem, gsem, wsem = (is0, is1), (gs0, gs1), (ws0, ws1)

    @pl.when(lax.axis_index("s") == 0)
    def _():
        pltpu.sync_copy(ptab_hbm, lut_s)

    plsc.subcore_barrier()

    def fire_idx(i, b):
        pltpu.async_copy(act_hbm.at[base + i], acts[b], isem[b])

    def drain_idx(b):
        pltpu.make_async_copy(act_hbm.at[0], acts[b], isem[b]).wait()

    def fire_gathers(b):
        for j in range(TVECS):
            pos = perm16 + j * (2 * TUP)
            a0 = plsc.load_gather(acts[b], [pos])
            a1 = plsc.load_gather(acts[b], [pos + 1])
            a2 = plsc.load_gather(acts[b], [pos + 2])
            a3 = plsc.load_gather(acts[b], [pos + 3])
            idx = ((a0 * NUM_ACTIONS + a1) * NUM_ACTIONS + a2) * NUM_ACTIONS + a3
            tidx[b][pl.ds(j * L, L)] = idx
        for t in range(ROW_T):
            pltpu.async_copy(
                lut_s.at[tidx[b].at[pl.ds(t * G, G)]],
                rows[b].at[pl.ds(t * G, G)],
                gsem[b],
            )

    def drain_gathers(b):
        pltpu.make_async_copy(
            ptab_hbm.at[pl.ds(0, CHUNK_T)], rows[b], gsem[b]
        ).wait()

    def fire_write(i, b):
        row0 = (base + i) * G
        for t in range(ROW_T):
            pltpu.async_copy(
                rows[b].at[pl.ds(t * G, G)],
                out_hbm.at[pl.ds(row0, G), pl.ds(t * ROW_W, ROW_W)],
                wsem[b],
            )

    def drain_write(b):
        pltpu.make_async_copy(
            ptab_hbm.at[pl.ds(0, CHUNK_T)], rows[b], wsem[b]
        ).wait()

    def slot(i, b, first, last):
        @pl.when(jnp.logical_not(first))
        def _():
            drain_write(b)
        drain_idx(b)
        fire_gathers(b)
        @pl.when(jnp.logical_not(last))
        def _():
            fire_idx(i + 2, b)
        @pl.when(i > 0)
        def _():
            drain_gathers(1 - b)
            fire_write(i - 1, 1 - b)

    fire_idx(0, 0)
    fire_idx(1, 1)

    @pl.loop(0, CPW, step=2)
    def _pair(c0):
        slot(c0, 0, c0 == 0, c0 + 2 >= CPW)
        slot(c0 + 1, 1, c0 == 0, c0 + 3 >= CPW)

    drain_gathers((CPW - 1) % 2)
    fire_write(CPW - 1, (CPW - 1) % 2)
    drain_write(0)
    drain_write(1)


def kernel(action, table):
    aidx = jnp.arange(NUM_ACTIONS**TUP, dtype=jnp.int32)
    parts = []
    for k in range(TUP):
        ak = (aidx // (NUM_ACTIONS ** (TUP - 1 - k))) % NUM_ACTIONS
        parts.append(jnp.take(table, ak, axis=0))
    ptab = jnp.concatenate(parts, axis=1)

    act2 = action.reshape(NCHUNKS, CHUNK_A)
    kfn = pl.kernel(
        _sc_body,
        out_type=jax.ShapeDtypeStruct((BATCH, OUT_W), jnp.float32),
        mesh=plsc.VectorSubcoreMesh(core_axis_name="c", subcore_axis_name="s"),
        compiler_params=pltpu.CompilerParams(
            needs_layout_passes=False, use_tc_tiling_on_sc=True
        ),
        scratch_types=[
            pltpu.VMEM_SHARED((NUM_ACTIONS**TUP, ROW_W), jnp.float32),
            pltpu.VMEM((CHUNK_A,), jnp.int32),
            pltpu.VMEM((CHUNK_A,), jnp.int32),
            pltpu.VMEM((CHUNK_T,), jnp.int32),
            pltpu.VMEM((CHUNK_T,), jnp.int32),
            pltpu.VMEM((CHUNK_T, ROW_W), jnp.float32),
            pltpu.VMEM((CHUNK_T, ROW_W), jnp.float32),
            pltpu.SemaphoreType.DMA,
            pltpu.SemaphoreType.DMA,
            pltpu.SemaphoreType.DMA,
            pltpu.SemaphoreType.DMA,
            pltpu.SemaphoreType.DMA,
            pltpu.SemaphoreType.DMA,
        ],
    )
    return kfn(act2, ptab)


# speedup vs baseline: 1.0284x; 1.0108x over previous
import jax
import jax.numpy as jnp
from jax import lax
from jax.experimental import pallas as pl
from jax.experimental.pallas import tpu as pltpu
from jax.experimental.pallas import tpu_sc as plsc

NUM_ACTIONS = 6
ACTION_DIM = 32
BATCH = 16384
HIST = 200

NC = 2
NS = 16
NW = NC * NS
L = 16

TUP = 4
ROW_T = HIST // TUP
G = 8
CHUNK_A = G * HIST
CHUNK_T = G * ROW_T
ROW_W = TUP * ACTION_DIM
OUT_W = HIST * ACTION_DIM
NCHUNKS = BATCH // G
CPW = NCHUNKS // NW
TVECS = CHUNK_T // L


def _sc_body(act_hbm, ptab_hbm, out_hbm, lut_s, a0_v, a1_v, t0_v, t1_v,
             r0_v, r1_v, is0, is1, gs0, gs1, ws0, ws1):
    wid = lax.axis_index("s") * NC + lax.axis_index("c")
    base = wid * CPW
    i16 = lax.iota(jnp.int32, 16)
    rvec = i16 % G
    cvec = (i16 // G) * TUP
    acts, tidx, rows = (a0_v, a1_v), (t0_v, t1_v), (r0_v, r1_v)
    isem, gsem, wsem = (is0, is1), (gs0, gs1), (ws0, ws1)

    @pl.when(lax.axis_index("s") == 0)
    def _():
        pltpu.sync_copy(ptab_hbm, lut_s)

    plsc.subcore_barrier()

    def fire_idx(i, b):
        pltpu.async_copy(act_hbm.at[base + i], acts[b], isem[b])

    def drain_idx(b):
        pltpu.make_async_copy(act_hbm.at[0], acts[b], isem[b]).wait()

    def fire_gathers(b):
        for j in range(TVECS):
            col = cvec + j * (2 * TUP)
            a0 = plsc.load_gather(acts[b], [rvec, col])
            a1 = plsc.load_gather(acts[b], [rvec, col + 1])
            a2 = plsc.load_gather(acts[b], [rvec, col + 2])
            a3 = plsc.load_gather(acts[b], [rvec, col + 3])
            idx = ((a0 * NUM_ACTIONS + a1) * NUM_ACTIONS + a2) * NUM_ACTIONS + a3
            tidx[b][pl.ds(j * L, L)] = idx
        for t in range(ROW_T):
            pltpu.async_copy(
                lut_s.at[tidx[b].at[pl.ds(t * G, G)]],
                rows[b].at[pl.ds(t * G, G)],
                gsem[b],
            )

    def drain_gathers(b):
        pltpu.make_async_copy(
            ptab_hbm.at[pl.ds(0, CHUNK_T)], rows[b], gsem[b]
        ).wait()

    def fire_write(i, b):
        row0 = (base + i) * G
        for t in range(ROW_T):
            pltpu.async_copy(
                rows[b].at[pl.ds(t * G, G)],
                out_hbm.at[pl.ds(row0, G), pl.ds(t * ROW_W, ROW_W)],
                wsem[b],
            )

    def drain_write(b):
        pltpu.make_async_copy(
            ptab_hbm.at[pl.ds(0, CHUNK_T)], rows[b], wsem[b]
        ).wait()

    def slot(i, b, first, last):
        @pl.when(jnp.logical_not(first))
        def _():
            drain_write(b)
        drain_idx(b)
        fire_gathers(b)
        @pl.when(jnp.logical_not(last))
        def _():
            fire_idx(i + 2, b)
        @pl.when(i > 0)
        def _():
            drain_gathers(1 - b)
            fire_write(i - 1, 1 - b)

    fire_idx(0, 0)
    fire_idx(1, 1)

    @pl.loop(0, CPW, step=2)
    def _pair(c0):
        slot(c0, 0, c0 == 0, c0 + 2 >= CPW)
        slot(c0 + 1, 1, c0 == 0, c0 + 3 >= CPW)

    drain_gathers((CPW - 1) % 2)
    fire_write(CPW - 1, (CPW - 1) % 2)
    drain_write(0)
    drain_write(1)


def kernel(action, table):
    aidx = jnp.arange(NUM_ACTIONS**TUP, dtype=jnp.int32)
    parts = []
    for k in range(TUP):
        ak = (aidx // (NUM_ACTIONS ** (TUP - 1 - k))) % NUM_ACTIONS
        parts.append(jnp.take(table, ak, axis=0))
    ptab = jnp.concatenate(parts, axis=1)

    act2 = action.reshape(NCHUNKS, G, HIST)
    kfn = pl.kernel(
        _sc_body,
        out_type=jax.ShapeDtypeStruct((BATCH, OUT_W), jnp.float32),
        mesh=plsc.VectorSubcoreMesh(core_axis_name="c", subcore_axis_name="s"),
        compiler_params=pltpu.CompilerParams(
            needs_layout_passes=False, use_tc_tiling_on_sc=True
        ),
        scratch_types=[
            pltpu.VMEM_SHARED((NUM_ACTIONS**TUP, ROW_W), jnp.float32),
            pltpu.VMEM((G, HIST), jnp.int32),
            pltpu.VMEM((G, HIST), jnp.int32),
            pltpu.VMEM((CHUNK_T,), jnp.int32),
            pltpu.VMEM((CHUNK_T,), jnp.int32),
            pltpu.VMEM((CHUNK_T, ROW_W), jnp.float32),
            pltpu.VMEM((CHUNK_T, ROW_W), jnp.float32),
            pltpu.SemaphoreType.DMA,
            pltpu.SemaphoreType.DMA,
            pltpu.SemaphoreType.DMA,
            pltpu.SemaphoreType.DMA,
            pltpu.SemaphoreType.DMA,
            pltpu.SemaphoreType.DMA,
        ],
    )
    return kfn(act2, ptab)
